# Initial kernel scaffold; baseline (speedup 1.0000x reference)
import jax
import jax.numpy as jnp
from jax import lax
from jax.experimental import pallas as pl
from jax.experimental.pallas import tpu as pltpu
from jax.experimental.pallas import tpu_sc as plsc

BATCH = 16384
SEQ = 200
D = 64
B = BATCH * SEQ
NC, NS = 2, 16
NW = NC * NS
B_PER_W = B // NW
CHUNK = 640
NCHUNKS = B_PER_W // CHUNK
assert B_PER_W % CHUNK == 0 and CHUNK % 8 == 0


def _lookup_body(idx_hbm, table_hbm, out_hbm, idx_v, rows_v, sem):
    wid = lax.axis_index("s") * NC + lax.axis_index("c")
    base = wid * B_PER_W

    def step(g, carry):
        off = base + g * CHUNK
        pltpu.sync_copy(idx_hbm.at[pl.ds(off, CHUNK)], idx_v)
        pltpu.async_copy(table_hbm.at[idx_v], rows_v, sem).wait()
        pltpu.sync_copy(rows_v, out_hbm.at[pl.ds(off, CHUNK)])
        return carry

    lax.fori_loop(0, NCHUNKS, step, 0)


@jax.jit
def _lookup(pos_ids_flat, table):
    mesh = plsc.VectorSubcoreMesh(
        core_axis_name="c", subcore_axis_name="s", num_cores=NC, num_subcores=NS
    )
    return pl.kernel(
        _lookup_body,
        out_type=jax.ShapeDtypeStruct((B, D), jnp.float32),
        mesh=mesh,
        scratch_types=[
            pltpu.VMEM((CHUNK,), jnp.int32),
            pltpu.VMEM((CHUNK, D), jnp.float32),
            pltpu.SemaphoreType.DMA,
        ],
    )(pos_ids_flat, table)


def kernel(pos_ids, table):
    flat = _lookup(pos_ids.reshape(B), table)
    return flat.reshape(BATCH, SEQ, D)


# speedup vs baseline: 4.6678x; 4.6678x over previous
import jax
import jax.numpy as jnp
from jax import lax
from jax.experimental import pallas as pl
from jax.experimental.pallas import tpu as pltpu
from jax.experimental.pallas import tpu_sc as plsc

BATCH = 16384
SEQ = 200
D = 64
B = BATCH * SEQ
PAIRS = B // 2
NC, NS = 2, 16
NW = NC * NS
P_PER_W = PAIRS // NW
CP = 320
NCHUNKS = P_PER_W // CP
L = 16
assert P_PER_W % CP == 0 and CP % L == 0


def _lookup_body(idx_hbm, pt_hbm, out_hbm, idxc_v, pidx_v, rows_v, sem):
    wid = lax.axis_index("s") * NC + lax.axis_index("c")
    base = wid * P_PER_W
    lanes = lax.iota(jnp.int32, L)

    def step(g, carry):
        off = base + g * CP
        pltpu.sync_copy(idx_hbm.at[pl.ds(2 * off, 2 * CP)], idxc_v)

        def pair_ids(t, c):
            even = plsc.load_gather(idxc_v, [2 * lanes + 2 * L * t])
            odd = plsc.load_gather(idxc_v, [2 * lanes + 2 * L * t + 1])
            pidx_v[pl.ds(L * t, L)] = even * 64 + odd
            return c

        lax.fori_loop(0, CP // L, pair_ids, 0)
        pltpu.async_copy(pt_hbm.at[pidx_v], rows_v, sem).wait()
        pltpu.sync_copy(rows_v, out_hbm.at[pl.ds(off, CP)])
        return carry

    lax.fori_loop(0, NCHUNKS, step, 0)


@jax.jit
def _lookup(pos_ids, table):
    idx_flat = pos_ids.reshape(B)
    tpad = jnp.pad(table, ((0, 64 - table.shape[0]), (0, 0)))
    left = jnp.broadcast_to(table[:, None, :], (50, 64, D))
    right = jnp.broadcast_to(tpad[None, :, :], (50, 64, D))
    pt = jnp.concatenate([left, right], axis=-1).reshape(3200, 2 * D)

    mesh = plsc.VectorSubcoreMesh(
        core_axis_name="c", subcore_axis_name="s", num_cores=NC, num_subcores=NS
    )
    flat = pl.kernel(
        _lookup_body,
        out_type=jax.ShapeDtypeStruct((PAIRS, 2 * D), jnp.float32),
        mesh=mesh,
        compiler_params=pltpu.CompilerParams(needs_layout_passes=False),
        scratch_types=[
            pltpu.VMEM((2 * CP,), jnp.int32),
            pltpu.VMEM((CP,), jnp.int32),
            pltpu.VMEM((CP, 2 * D), jnp.float32),
            pltpu.SemaphoreType.DMA,
        ],
    )(idx_flat, pt)
    return flat.reshape(BATCH, SEQ, D)


def kernel(pos_ids, table):
    return _lookup(pos_ids, table)


# speedup vs baseline: 4.8740x; 1.0442x over previous
import jax
import jax.numpy as jnp
from jax import lax
from jax.experimental import pallas as pl
from jax.experimental.pallas import tpu as pltpu
from jax.experimental.pallas import tpu_sc as plsc

BATCH = 16384
SEQ = 200
D = 64
B = BATCH * SEQ
PAIRS = B // 2
NC, NS = 2, 16
NW = NC * NS
P_PER_W = PAIRS // NW
CP = 320
NCHUNKS = P_PER_W // CP
L = 16
assert P_PER_W % CP == 0 and CP % L == 0 and NCHUNKS % 2 == 0


def _lookup_body(idx_hbm, pt_hbm, out_hbm,
                 idxc0, idxc1, pidx0, pidx1, rows0, rows1,
                 sem_i0, sem_i1, sem_g0, sem_g1, sem_o0, sem_o1):
    wid = lax.axis_index("s") * NC + lax.axis_index("c")
    base = wid * P_PER_W
    lanes = lax.iota(jnp.int32, L)
    idxc_v = (idxc0, idxc1)
    pidx_v = (pidx0, pidx1)
    rows_v = (rows0, rows1)
    sem_i = (sem_i0, sem_i1)
    sem_g = (sem_g0, sem_g1)
    sem_o = (sem_o0, sem_o1)

    def start_idx(g, b):
        off = base + g * CP
        pltpu.async_copy(idx_hbm.at[pl.ds(2 * off, 2 * CP)],
                         idxc_v[b], sem_i[b])

    start_idx(0, 0)
    start_idx(1, 1)

    def outer(g2, carry):
        for b in range(2):
            g = 2 * g2 + b
            pltpu.make_async_copy(idx_hbm.at[pl.ds(0, 2 * CP)],
                                  idxc_v[b], sem_i[b]).wait()

            def pair_ids(t, c):
                even = plsc.load_gather(idxc_v[b], [2 * lanes + 2 * L * t])
                odd = plsc.load_gather(idxc_v[b], [2 * lanes + 2 * L * t + 1])
                pidx_v[b][pl.ds(L * t, L)] = even * 64 + odd
                return c

            lax.fori_loop(0, CP // L, pair_ids, 0)

            @pl.when(g2 > 0)
            def _():
                pltpu.make_async_copy(rows_v[b],
                                      out_hbm.at[pl.ds(0, CP)], sem_o[b]).wait()

            gather = pltpu.async_copy(pt_hbm.at[pidx_v[b]],
                                      rows_v[b], sem_g[b])

            @pl.when(g2 < NCHUNKS // 2 - 1)
            def _():
                start_idx(g + 2, b)

            gather.wait()
            off = base + g * CP
            pltpu.async_copy(rows_v[b], out_hbm.at[pl.ds(off, CP)], sem_o[b])
        return carry

    lax.fori_loop(0, NCHUNKS // 2, outer, 0)
    for b in range(2):
        pltpu.make_async_copy(rows_v[b],
                              out_hbm.at[pl.ds(0, CP)], sem_o[b]).wait()


@jax.jit
def _lookup(pos_ids, table):
    idx_flat = pos_ids.reshape(B)
    tpad = jnp.pad(table, ((0, 64 - table.shape[0]), (0, 0)))
    left = jnp.broadcast_to(table[:, None, :], (50, 64, D))
    right = jnp.broadcast_to(tpad[None, :, :], (50, 64, D))
    pt = jnp.concatenate([left, right], axis=-1).reshape(3200, 2 * D)

    mesh = plsc.VectorSubcoreMesh(
        core_axis_name="c", subcore_axis_name="s", num_cores=NC, num_subcores=NS
    )
    flat = pl.kernel(
        _lookup_body,
        out_type=jax.ShapeDtypeStruct((PAIRS, 2 * D), jnp.float32),
        mesh=mesh,
        compiler_params=pltpu.CompilerParams(needs_layout_passes=False),
        scratch_types=[
            pltpu.VMEM((2 * CP,), jnp.int32),
            pltpu.VMEM((2 * CP,), jnp.int32),
            pltpu.VMEM((CP,), jnp.int32),
            pltpu.VMEM((CP,), jnp.int32),
            pltpu.VMEM((CP, 2 * D), jnp.float32),
            pltpu.VMEM((CP, 2 * D), jnp.float32),
        ] + [pltpu.SemaphoreType.DMA] * 6,
    )(idx_flat, pt)
    return flat.reshape(BATCH, SEQ, D)


def kernel(pos_ids, table):
    return _lookup(pos_ids, table)


# speedup vs baseline: 7.4621x; 1.5310x over previous
import jax
import jax.numpy as jnp
from jax import lax
from jax.experimental import pallas as pl
from jax.experimental.pallas import tpu as pltpu
from jax.experimental.pallas import tpu_sc as plsc

BATCH = 16384
SEQ = 200
D = 64
V = 50
NC, NS = 2, 16
NW = NC * NS
K = BATCH // NW
L = 16


def _lookup_body(idxt_hbm, ttab_hbm, out_hbm,
                 ttab_v, idx0, idx1, stage0, stage1,
                 sem_t, sem_i0, sem_i1, sem_o0, sem_o1):
    wid = lax.axis_index("s") * NC + lax.axis_index("c")
    b0 = wid * K
    lanes = lax.iota(jnp.int32, L)
    idx_v = (idx0, idx1)
    stage_v = (stage0, stage1)
    sem_i = (sem_i0, sem_i1)
    sem_o = (sem_o0, sem_o1)

    pltpu.async_copy(ttab_hbm, ttab_v, sem_t).wait()

    def start_idx(s, b):
        pltpu.async_copy(idxt_hbm.at[s, pl.ds(b0, K)], idx_v[b], sem_i[b])

    start_idx(0, 0)
    start_idx(1, 1)

    def outer(s2, carry):
        for b in range(2):
            s = 2 * s2 + b
            pltpu.make_async_copy(idxt_hbm.at[0, pl.ds(0, K)],
                                  idx_v[b], sem_i[b]).wait()

            @pl.when(s2 > 0)
            def _():
                pltpu.make_async_copy(
                    stage_v[b], out_hbm.at[0, :, pl.ds(0, K)], sem_o[b]).wait()

            def lane_group(t, c):
                v0 = plsc.load_gather(idx_v[b], [lanes + L * t])

                def dstep(d, v):
                    stage_v[b][d, pl.ds(L * t, L)] = plsc.load_gather(ttab_v, [v])
                    return v + D
                lax.fori_loop(0, D, dstep, v0)
                return c

            lax.fori_loop(0, K // L, lane_group, 0)

            @pl.when(s2 < SEQ // 2 - 1)
            def _():
                start_idx(s + 2, b)

            pltpu.async_copy(stage_v[b], out_hbm.at[s, :, pl.ds(b0, K)],
                             sem_o[b])
        return carry

    lax.fori_loop(0, SEQ // 2, outer, 0)
    for b in range(2):
        pltpu.make_async_copy(stage_v[b],
                              out_hbm.at[0, :, pl.ds(0, K)], sem_o[b]).wait()


@jax.jit
def _lookup(pos_ids, table):
    idxt = pos_ids.T
    ttab_flat = jnp.pad(table, ((0, D - V), (0, 0))).T.reshape(D * D)

    mesh = plsc.VectorSubcoreMesh(
        core_axis_name="c", subcore_axis_name="s", num_cores=NC, num_subcores=NS
    )
    out_t = pl.kernel(
        _lookup_body,
        out_type=jax.ShapeDtypeStruct((SEQ, D, BATCH), jnp.float32),
        mesh=mesh,
        compiler_params=pltpu.CompilerParams(needs_layout_passes=False),
        scratch_types=[
            pltpu.VMEM((D * D,), jnp.float32),
            pltpu.VMEM((K,), jnp.int32),
            pltpu.VMEM((K,), jnp.int32),
            pltpu.VMEM((D, K), jnp.float32),
            pltpu.VMEM((D, K), jnp.float32),
        ] + [pltpu.SemaphoreType.DMA] * 5,
    )(idxt, ttab_flat)
    return out_t.transpose(2, 0, 1)


def kernel(pos_ids, table):
    return _lookup(pos_ids, table)


# speedup vs baseline: 7.6645x; 1.0271x over previous
import jax
import jax.numpy as jnp
from jax import lax
from jax.experimental import pallas as pl
from jax.experimental.pallas import tpu as pltpu
from jax.experimental.pallas import tpu_sc as plsc

BATCH = 16384
SEQ = 200
D = 64
V = 50
NC, NS = 2, 16
NW = NC * NS
K = BATCH // NW
L = 16


def _lookup_body(idxt_hbm, ttab_hbm, out_hbm,
                 ttab_v, idx0, idx1, stage0, stage1,
                 sem_t, sem_i0, sem_i1, sem_o0, sem_o1):
    wid = lax.axis_index("s") * NC + lax.axis_index("c")
    b0 = wid * K
    lanes = lax.iota(jnp.int32, L)
    idx_v = (idx0, idx1)
    stage_v = (stage0, stage1)
    sem_i = (sem_i0, sem_i1)
    sem_o = (sem_o0, sem_o1)

    pltpu.async_copy(ttab_hbm, ttab_v, sem_t).wait()

    def start_idx(s, b):
        pltpu.async_copy(idxt_hbm.at[s, pl.ds(b0, K)], idx_v[b], sem_i[b])

    start_idx(0, 0)
    start_idx(1, 1)

    def outer(s2, carry):
        for b in range(2):
            s = 2 * s2 + b
            pltpu.make_async_copy(idxt_hbm.at[0, pl.ds(0, K)],
                                  idx_v[b], sem_i[b]).wait()

            @pl.when(s2 > 0)
            def _():
                pltpu.make_async_copy(
                    stage_v[b], out_hbm.at[0, :, pl.ds(0, K)], sem_o[b]).wait()

            def lane_group(t, c):
                v0 = plsc.load_gather(idx_v[b], [lanes + L * t])
                for d in range(D):
                    stage_v[b][d, pl.ds(L * t, L)] = plsc.load_gather(
                        ttab_v, [v0 + jnp.int32(d * D)])
                return c

            lax.fori_loop(0, K // L, lane_group, 0)

            @pl.when(s2 < SEQ // 2 - 1)
            def _():
                start_idx(s + 2, b)

            pltpu.async_copy(stage_v[b], out_hbm.at[s, :, pl.ds(b0, K)],
                             sem_o[b])
        return carry

    lax.fori_loop(0, SEQ // 2, outer, 0)
    for b in range(2):
        pltpu.make_async_copy(stage_v[b],
                              out_hbm.at[0, :, pl.ds(0, K)], sem_o[b]).wait()


@jax.jit
def _lookup(pos_ids, table):
    idxt = pos_ids.T
    ttab_flat = jnp.pad(table, ((0, D - V), (0, 0))).T.reshape(D * D)

    mesh = plsc.VectorSubcoreMesh(
        core_axis_name="c", subcore_axis_name="s", num_cores=NC, num_subcores=NS
    )
    out_t = pl.kernel(
        _lookup_body,
        out_type=jax.ShapeDtypeStruct((SEQ, D, BATCH), jnp.float32),
        mesh=mesh,
        compiler_params=pltpu.CompilerParams(needs_layout_passes=False),
        scratch_types=[
            pltpu.VMEM((D * D,), jnp.float32),
            pltpu.VMEM((K,), jnp.int32),
            pltpu.VMEM((K,), jnp.int32),
            pltpu.VMEM((D, K), jnp.float32),
            pltpu.VMEM((D, K), jnp.float32),
        ] + [pltpu.SemaphoreType.DMA] * 5,
    )(idxt, ttab_flat)
    return out_t.transpose(2, 0, 1)


def kernel(pos_ids, table):
    return _lookup(pos_ids, table)


# speedup vs baseline: 15.3199x; 1.9988x over previous
import jax
import jax.numpy as jnp
from jax import lax
from jax.experimental import pallas as pl
from jax.experimental.pallas import tpu as pltpu
from jax.experimental.pallas import tpu_sc as plsc

BATCH = 16384
SEQ = 200
D = 64
V = 50
NC, NS = 2, 16
NW = NC * NS
K = BATCH // NW
L = 16


def _lookup_body(idxt_hbm, ttab_hbm, out_hbm,
                 ttab_v, idx0, idx1, stage0, stage1,
                 sem_t, sem_i0, sem_i1, sem_o0, sem_o1):
    wid = lax.axis_index("s") * NC + lax.axis_index("c")
    b0 = wid * K
    lanes = lax.iota(jnp.int32, L)
    idx_v = (idx0, idx1)
    stage_v = (stage0, stage1)
    sem_i = (sem_i0, sem_i1)
    sem_o = (sem_o0, sem_o1)

    pltpu.async_copy(ttab_hbm, ttab_v, sem_t).wait()

    def start_idx(s, b):
        pltpu.async_copy(idxt_hbm.at[s, pl.ds(b0, K)], idx_v[b], sem_i[b])

    start_idx(0, 0)
    start_idx(1, 1)

    def outer(s2, carry):
        for b in range(2):
            s = 2 * s2 + b
            pltpu.make_async_copy(idxt_hbm.at[0, pl.ds(0, K)],
                                  idx_v[b], sem_i[b]).wait()

            @pl.when(s2 > 0)
            def _():
                pltpu.make_async_copy(
                    stage_v[b], out_hbm.at[0, :, pl.ds(0, K)], sem_o[b]).wait()

            @plsc.parallel_loop(0, K // L, unroll=2)
            def lane_group(t):
                v0 = plsc.load_gather(idx_v[b], [lanes + L * t])
                for d in range(D):
                    stage_v[b][d, pl.ds(L * t, L)] = plsc.load_gather(
                        ttab_v, [v0 + jnp.int32(d * D)])

            @pl.when(s2 < SEQ // 2 - 1)
            def _():
                start_idx(s + 2, b)

            pltpu.async_copy(stage_v[b], out_hbm.at[s, :, pl.ds(b0, K)],
                             sem_o[b])
        return carry

    lax.fori_loop(0, SEQ // 2, outer, 0)
    for b in range(2):
        pltpu.make_async_copy(stage_v[b],
                              out_hbm.at[0, :, pl.ds(0, K)], sem_o[b]).wait()


@jax.jit
def _lookup(pos_ids, table):
    idxt = pos_ids.T
    ttab_flat = jnp.pad(table, ((0, D - V), (0, 0))).T.reshape(D * D)

    mesh = plsc.VectorSubcoreMesh(
        core_axis_name="c", subcore_axis_name="s", num_cores=NC, num_subcores=NS
    )
    out_t = pl.kernel(
        _lookup_body,
        out_type=jax.ShapeDtypeStruct((SEQ, D, BATCH), jnp.float32),
        mesh=mesh,
        compiler_params=pltpu.CompilerParams(needs_layout_passes=False),
        scratch_types=[
            pltpu.VMEM((D * D,), jnp.float32),
            pltpu.VMEM((K,), jnp.int32),
            pltpu.VMEM((K,), jnp.int32),
            pltpu.VMEM((D, K), jnp.float32),
            pltpu.VMEM((D, K), jnp.float32),
        ] + [pltpu.SemaphoreType.DMA] * 5,
    )(idxt, ttab_flat)
    return out_t.transpose(2, 0, 1)


def kernel(pos_ids, table):
    return _lookup(pos_ids, table)


# speedup vs baseline: 23.1722x; 1.5126x over previous
import jax
import jax.numpy as jnp
from jax import lax
from jax.experimental import pallas as pl
from jax.experimental.pallas import tpu as pltpu
from jax.experimental.pallas import tpu_sc as plsc

BATCH = 16384
SEQ = 200
D = 64
V = 50
NC, NS = 2, 16
NW = NC * NS
K = BATCH // NW
L = 16


def _lookup_body(idxt_hbm, ttab_hbm, out_hbm,
                 ttab_v, idx0, idx1, stage0, stage1,
                 sem_t, sem_i0, sem_i1, sem_o0, sem_o1):
    wid = lax.axis_index("s") * NC + lax.axis_index("c")
    b0 = wid * K
    lanes = lax.iota(jnp.int32, L)
    idx_v = (idx0, idx1)
    stage_v = (stage0, stage1)
    sem_i = (sem_i0, sem_i1)
    sem_o = (sem_o0, sem_o1)

    pltpu.async_copy(ttab_hbm, ttab_v, sem_t).wait()

    def start_idx(s, b):
        pltpu.async_copy(idxt_hbm.at[s, pl.ds(b0, K)], idx_v[b], sem_i[b])

    start_idx(0, 0)
    start_idx(1, 1)

    def outer(s2, carry):
        for b in range(2):
            s = 2 * s2 + b
            pltpu.make_async_copy(idxt_hbm.at[0, pl.ds(0, K)],
                                  idx_v[b], sem_i[b]).wait()

            @pl.when(s2 > 0)
            def _():
                pltpu.make_async_copy(
                    stage_v[b], out_hbm.at[0, :, pl.ds(0, K)], sem_o[b]).wait()

            @plsc.parallel_loop(0, K // L, unroll=4)
            def lane_group(t):
                v0 = plsc.load_gather(idx_v[b], [lanes + L * t])
                for d in range(D):
                    stage_v[b][d, pl.ds(L * t, L)] = plsc.load_gather(
                        ttab_v, [v0 + jnp.int32(d * D)])

            @pl.when(s2 < SEQ // 2 - 1)
            def _():
                start_idx(s + 2, b)

            pltpu.async_copy(stage_v[b], out_hbm.at[s, :, pl.ds(b0, K)],
                             sem_o[b])
        return carry

    lax.fori_loop(0, SEQ // 2, outer, 0)
    for b in range(2):
        pltpu.make_async_copy(stage_v[b],
                              out_hbm.at[0, :, pl.ds(0, K)], sem_o[b]).wait()


@jax.jit
def _lookup(pos_ids, table):
    idxt = pos_ids.T
    ttab_flat = jnp.pad(table, ((0, D - V), (0, 0))).T.reshape(D * D)

    mesh = plsc.VectorSubcoreMesh(
        core_axis_name="c", subcore_axis_name="s", num_cores=NC, num_subcores=NS
    )
    out_t = pl.kernel(
        _lookup_body,
        out_type=jax.ShapeDtypeStruct((SEQ, D, BATCH), jnp.float32),
        mesh=mesh,
        compiler_params=pltpu.CompilerParams(needs_layout_passes=False),
        scratch_types=[
            pltpu.VMEM((D * D,), jnp.float32),
            pltpu.VMEM((K,), jnp.int32),
            pltpu.VMEM((K,), jnp.int32),
            pltpu.VMEM((D, K), jnp.float32),
            pltpu.VMEM((D, K), jnp.float32),
        ] + [pltpu.SemaphoreType.DMA] * 5,
    )(idxt, ttab_flat)
    return out_t.transpose(2, 0, 1)


def kernel(pos_ids, table):
    return _lookup(pos_ids, table)
